# Initial kernel scaffold; baseline (speedup 1.0000x reference)
import jax
import jax.numpy as jnp
from jax.experimental import pallas as pl


def kernel(x, table):
    raise NotImplementedError("write your pallas kernel here")


# speedup vs baseline: 2.9931x; 2.9931x over previous
import functools

import jax
import jax.numpy as jnp
from jax import lax
from jax.experimental import pallas as pl
from jax.experimental.pallas import tpu as pltpu
from jax.experimental.pallas import tpu_sc as plsc

H_DIM = 128
NUM_WORKERS = 32
CHUNK = 128


def _gather_kernel(idx_hbm, table_hbm, out_hbm, idx_v, rows_v, sem):
    wid = lax.axis_index("s") * 2 + lax.axis_index("c")
    n_chunks = idx_v.shape[0]
    pltpu.sync_copy(idx_hbm.at[wid], idx_v)

    def body(j, carry):
        pltpu.async_copy(table_hbm.at[idx_v.at[j]], rows_v, sem).wait()
        base = (wid * n_chunks + j) * CHUNK
        pltpu.sync_copy(rows_v, out_hbm.at[pl.ds(base, CHUNK)])
        return carry

    lax.fori_loop(0, n_chunks, body, 0)


def kernel(x, table):
    batch, n_fields = x.shape
    total = batch * n_fields
    n_chunks = total // (NUM_WORKERS * CHUNK)
    idx = x.reshape(NUM_WORKERS, n_chunks, CHUNK).astype(jnp.int32)

    mesh = plsc.VectorSubcoreMesh(core_axis_name="c", subcore_axis_name="s")
    run = functools.partial(
        pl.kernel,
        mesh=mesh,
        out_type=jax.ShapeDtypeStruct((total, H_DIM), jnp.float32),
        scratch_types=[
            pltpu.VMEM((n_chunks, CHUNK), jnp.int32),
            pltpu.VMEM((CHUNK, H_DIM), jnp.float32),
            pltpu.SemaphoreType.DMA,
        ],
    )(_gather_kernel)

    out = run(idx, table)
    return out.reshape(batch, n_fields, H_DIM)


# speedup vs baseline: 3.3615x; 1.1231x over previous
import functools

import jax
import jax.numpy as jnp
from jax import lax
from jax.experimental import pallas as pl
from jax.experimental.pallas import tpu as pltpu
from jax.experimental.pallas import tpu_sc as plsc

H_DIM = 128
NUM_WORKERS = 32
CHUNK = 128
NBUF = 4


def _gather_kernel(idx_hbm, table_hbm, out_hbm, idx_v, rows_v, gsems, ssems):
    wid = lax.axis_index("s") * 2 + lax.axis_index("c")
    n_chunks = idx_v.shape[0]
    n_groups = n_chunks // NBUF
    pltpu.sync_copy(idx_hbm.at[wid], idx_v)

    def gather_copy(b, chunk):
        return pltpu.make_async_copy(
            table_hbm.at[idx_v.at[chunk]], rows_v.at[b], gsems.at[b]
        )

    def store_copy(b, chunk):
        base = (wid * n_chunks + chunk) * CHUNK
        return pltpu.make_async_copy(
            rows_v.at[b], out_hbm.at[pl.ds(base, CHUNK)], ssems.at[b]
        )

    for b in range(NBUF):
        gather_copy(b, b).start()

    def body(g, carry):
        for b in range(NBUF):
            chunk = g * NBUF + b
            gather_copy(b, chunk).wait()
            store_copy(b, chunk).start()
        for b in range(NBUF):
            chunk = g * NBUF + b
            store_copy(b, chunk).wait()

            @pl.when(g + 1 < n_groups)
            def _():
                gather_copy(b, (g + 1) * NBUF + b).start()

        return carry

    lax.fori_loop(0, n_groups, body, 0)


def kernel(x, table):
    batch, n_fields = x.shape
    total = batch * n_fields
    n_chunks = total // (NUM_WORKERS * CHUNK)
    idx = x.reshape(NUM_WORKERS, n_chunks, CHUNK).astype(jnp.int32)

    mesh = plsc.VectorSubcoreMesh(core_axis_name="c", subcore_axis_name="s")
    run = functools.partial(
        pl.kernel,
        mesh=mesh,
        out_type=jax.ShapeDtypeStruct((total, H_DIM), jnp.float32),
        scratch_types=[
            pltpu.VMEM((n_chunks, CHUNK), jnp.int32),
            pltpu.VMEM((NBUF, CHUNK, H_DIM), jnp.float32),
            pltpu.SemaphoreType.DMA((NBUF,)),
            pltpu.SemaphoreType.DMA((NBUF,)),
        ],
    )(_gather_kernel)

    out = run(idx, table)
    return out.reshape(batch, n_fields, H_DIM)
